# baseline (speedup 1.0000x reference)
import jax
import jax.numpy as jnp
from jax.experimental import pallas as pl
from jax.experimental.pallas import tpu as pltpu

B = 32
T = 50
H = 1024
K = 64
D = 20
R = B * T
RC = 400
G1 = R // RC
WQ = 4 * H // G1
SPG = 10


def _precompute_kernel(x1_ref, x2_ref, sk_ref, wm1_ref, bm1_ref, wm2_ref,
                       bm2_ref, sel1_ref, sel2_ref, expc_ref, kiota_ref,
                       wflat_ref, benc_ref, wih_ref, bg_ref, whh_ref,
                       xg_ref, whht_ref, wiht_s, wflat_s):
    i = pl.program_id(0)

    @pl.when(i == 0)
    def _():
        wiht_s[...] = wih_ref[...].astype(jnp.bfloat16).T
        wflat_s[...] = wflat_ref[...].astype(jnp.bfloat16)

    whht_ref[...] = whh_ref[...].astype(jnp.bfloat16).T

    f1 = jnp.dot(x1_ref[...], wm1_ref[...],
                 preferred_element_type=jnp.float32) + bm1_ref[...]
    f2 = jnp.dot(x2_ref[...], wm2_ref[...],
                 preferred_element_type=jnp.float32) + bm2_ref[...]
    tiled = (jnp.dot(f1.astype(jnp.bfloat16), sel1_ref[...],
                     preferred_element_type=jnp.float32) +
             jnp.dot(f2.astype(jnp.bfloat16), sel2_ref[...],
                     preferred_element_type=jnp.float32))
    sk = sk_ref[...]
    p = jnp.where(expc_ref[...] == sk, tiled, 0.0)
    onehot = (kiota_ref[...] == sk).astype(jnp.bfloat16)
    tmp = (jnp.dot(p.astype(jnp.bfloat16), wflat_s[...],
                   preferred_element_type=jnp.float32) +
           jnp.dot(onehot, benc_ref[...],
                   preferred_element_type=jnp.float32))
    xg_ref[...] = jnp.dot(tmp.astype(jnp.bfloat16), wiht_s[...],
                          preferred_element_type=jnp.float32
                          ).astype(jnp.bfloat16)


def _recurrent_kernel(xg_ref, h0_ref, c0_ref, whh_ref, wdec_ref, bdec_ref,
                      bg_ref, out_ref, hout_ref, cout_ref, h_scr, c_scr):
    t = pl.program_id(0)

    @pl.when(t == 0)
    def _():
        h_scr[...] = h0_ref[...]
        c_scr[...] = c0_ref[...]

    h = h_scr[...]
    c = c_scr[...]
    bg = bg_ref[...]
    for step in range(SPG):
        gates = (xg_ref[step].astype(jnp.float32) + bg +
                 jnp.dot(h.astype(jnp.bfloat16), whh_ref[...],
                         preferred_element_type=jnp.float32))
        i_g = gates[:, 0 * H:1 * H]
        f_g = gates[:, 1 * H:2 * H]
        g_g = gates[:, 2 * H:3 * H]
        o_g = gates[:, 3 * H:4 * H]
        c = jax.nn.sigmoid(f_g) * c + jax.nn.sigmoid(i_g) * jnp.tanh(g_g)
        h = jax.nn.sigmoid(o_g) * jnp.tanh(c)
        out_ref[:, step * K:(step + 1) * K] = jnp.dot(
            h.astype(jnp.bfloat16), wdec_ref[...],
            preferred_element_type=jnp.float32) + bdec_ref[...]
    h_scr[...] = h
    c_scr[...] = c
    hout_ref[...] = h
    cout_ref[...] = c


@jax.jit
def kernel(input_1, input_2, h0, c0, routers_info, W_m1, b_m1, W_m2, b_m2,
           W_enc, b_enc, W_ih, W_hh, b_ih, b_hh, W_dec, b_dec):
    bf16 = jnp.bfloat16
    x1 = input_1.transpose(1, 0, 2).reshape(R, 2)
    x2 = input_2.transpose(1, 0, 2).reshape(R, 1)
    sk = routers_info.T.reshape(R, 1)
    w_flat = W_enc.reshape(K * D, H)
    benc_b = b_enc.astype(bf16)
    wdec_t = W_dec.astype(bf16).T
    b_gates = (b_ih + b_hh).reshape(1, 4 * H)
    cols = jnp.arange(K * D, dtype=jnp.int32)
    expc = (cols // D).reshape(1, K * D)
    dmod = cols % D
    sel = (dmod[None, :] == jnp.arange(D, dtype=jnp.int32)[:, None])
    sel = sel.astype(bf16)
    sel1, sel2 = sel[:10], sel[10:]
    kiota = jnp.arange(K, dtype=jnp.int32).reshape(1, K)

    ci = lambda i: (i, 0)
    cc = lambda i: (0, 0)
    xg, whh_t = pl.pallas_call(
        _precompute_kernel,
        grid=(G1,),
        in_specs=[
            pl.BlockSpec((RC, 2), ci),
            pl.BlockSpec((RC, 1), ci),
            pl.BlockSpec((RC, 1), ci),
            pl.BlockSpec((2, 10), cc),
            pl.BlockSpec((1, 10), cc),
            pl.BlockSpec((1, 10), cc),
            pl.BlockSpec((1, 10), cc),
            pl.BlockSpec((10, K * D), cc),
            pl.BlockSpec((10, K * D), cc),
            pl.BlockSpec((1, K * D), cc),
            pl.BlockSpec((1, K), cc),
            pl.BlockSpec((K * D, H), cc),
            pl.BlockSpec((K, H), cc),
            pl.BlockSpec((4 * H, H), cc),
            pl.BlockSpec((1, 4 * H), cc),
            pl.BlockSpec((WQ, H), ci),
        ],
        out_specs=[
            pl.BlockSpec((RC, 4 * H), ci),
            pl.BlockSpec((H, WQ), lambda i: (0, i)),
        ],
        out_shape=[
            jax.ShapeDtypeStruct((R, 4 * H), bf16),
            jax.ShapeDtypeStruct((H, 4 * H), bf16),
        ],
        scratch_shapes=[
            pltpu.VMEM((H, 4 * H), bf16),
            pltpu.VMEM((K * D, H), bf16),
        ],
    )(x1, x2, sk, W_m1.T, b_m1.reshape(1, 10), W_m2.T, b_m2.reshape(1, 10),
      sel1, sel2, expc, kiota, w_flat, benc_b, W_ih, b_gates, W_hh)

    xg3 = xg.reshape(T, B, 4 * H)
    out3, h_t, c_t = pl.pallas_call(
        _recurrent_kernel,
        grid=(T // SPG,),
        in_specs=[
            pl.BlockSpec((SPG, B, 4 * H), lambda t: (t, 0, 0)),
            pl.BlockSpec((B, H), lambda t: (0, 0)),
            pl.BlockSpec((B, H), lambda t: (0, 0)),
            pl.BlockSpec((H, 4 * H), lambda t: (0, 0)),
            pl.BlockSpec((H, K), lambda t: (0, 0)),
            pl.BlockSpec((1, K), lambda t: (0, 0)),
            pl.BlockSpec((1, 4 * H), lambda t: (0, 0)),
        ],
        out_specs=[
            pl.BlockSpec((B, SPG * K), lambda t: (0, t)),
            pl.BlockSpec((B, H), lambda t: (0, 0)),
            pl.BlockSpec((B, H), lambda t: (0, 0)),
        ],
        out_shape=[
            jax.ShapeDtypeStruct((B, T * K), jnp.float32),
            jax.ShapeDtypeStruct((B, H), jnp.float32),
            jax.ShapeDtypeStruct((B, H), jnp.float32),
        ],
        scratch_shapes=[
            pltpu.VMEM((B, H), jnp.float32),
            pltpu.VMEM((B, H), jnp.float32),
        ],
    )(xg3, h0, c0, whh_t, wdec_t, b_dec.reshape(1, K), b_gates)

    output = out3.reshape(B * T, K)
    return (output, h_t, c_t)


# speedup vs baseline: 1.0649x; 1.0649x over previous
import jax
import jax.numpy as jnp
from jax.experimental import pallas as pl
from jax.experimental.pallas import tpu as pltpu

B = 32
T = 50
H = 1024
K = 64
D = 20
R = B * T
RC = 400
G1 = R // RC
WQ = 4 * H // G1
SPG = 10


def _precompute_kernel(x1_ref, x2_ref, sk_ref, wm1_ref, bm1_ref, wm2_ref,
                       bm2_ref, sel1_ref, sel2_ref, expc_ref, kiota_ref,
                       wflat_ref, benc_ref, wih_ref, bg_ref, whh_ref,
                       xg_ref, whht_ref, wiht_s):
    i = pl.program_id(0)

    @pl.when(i == 0)
    def _():
        wiht_s[...] = wih_ref[...].astype(jnp.bfloat16).T

    whht_ref[...] = whh_ref[...].astype(jnp.bfloat16).T

    f1 = jnp.dot(x1_ref[...], wm1_ref[...],
                 preferred_element_type=jnp.float32) + bm1_ref[...]
    f2 = jnp.dot(x2_ref[...], wm2_ref[...],
                 preferred_element_type=jnp.float32) + bm2_ref[...]
    tiled = (jnp.dot(f1.astype(jnp.bfloat16), sel1_ref[...],
                     preferred_element_type=jnp.float32) +
             jnp.dot(f2.astype(jnp.bfloat16), sel2_ref[...],
                     preferred_element_type=jnp.float32))
    sk = sk_ref[...]
    p = jnp.where(expc_ref[...] == sk, tiled, 0.0)
    onehot = (kiota_ref[...] == sk).astype(jnp.bfloat16)
    tmp = (jnp.dot(p.astype(jnp.bfloat16), wflat_ref[...],
                   preferred_element_type=jnp.float32) +
           jnp.dot(onehot, benc_ref[...],
                   preferred_element_type=jnp.float32))
    xg_ref[...] = jnp.dot(tmp.astype(jnp.bfloat16), wiht_s[...],
                          preferred_element_type=jnp.float32
                          ).astype(jnp.bfloat16)


def _recurrent_kernel(xg_ref, h0_ref, c0_ref, whh_ref, wdec_ref, bdec_ref,
                      bg_ref, out_ref, hout_ref, cout_ref, h_scr, c_scr):
    t = pl.program_id(0)

    @pl.when(t == 0)
    def _():
        h_scr[...] = h0_ref[...]
        c_scr[...] = c0_ref[...]

    h = h_scr[...]
    c = c_scr[...]
    bg = bg_ref[...]
    for step in range(SPG):
        gates = (xg_ref[step].astype(jnp.float32) + bg +
                 jnp.dot(h.astype(jnp.bfloat16), whh_ref[...],
                         preferred_element_type=jnp.float32))
        i_g = gates[:, 0 * H:1 * H]
        f_g = gates[:, 1 * H:2 * H]
        g_g = gates[:, 2 * H:3 * H]
        o_g = gates[:, 3 * H:4 * H]
        c = jax.nn.sigmoid(f_g) * c + jax.nn.sigmoid(i_g) * jnp.tanh(g_g)
        h = jax.nn.sigmoid(o_g) * jnp.tanh(c)
        out_ref[:, step * K:(step + 1) * K] = jnp.dot(
            h.astype(jnp.bfloat16), wdec_ref[...],
            preferred_element_type=jnp.float32) + bdec_ref[...]
    h_scr[...] = h
    c_scr[...] = c
    hout_ref[...] = h
    cout_ref[...] = c


@jax.jit
def kernel(input_1, input_2, h0, c0, routers_info, W_m1, b_m1, W_m2, b_m2,
           W_enc, b_enc, W_ih, W_hh, b_ih, b_hh, W_dec, b_dec):
    bf16 = jnp.bfloat16
    x1 = input_1.transpose(1, 0, 2).reshape(R, 2)
    x2 = input_2.transpose(1, 0, 2).reshape(R, 1)
    sk = routers_info.T.reshape(R, 1)
    w_flat = W_enc.reshape(K * D, H).astype(bf16)
    benc_b = b_enc.astype(bf16)
    wdec_t = W_dec.astype(bf16).T
    b_gates = (b_ih + b_hh).reshape(1, 4 * H)
    cols = jnp.arange(K * D, dtype=jnp.int32)
    expc = (cols // D).reshape(1, K * D)
    dmod = cols % D
    sel = (dmod[None, :] == jnp.arange(D, dtype=jnp.int32)[:, None])
    sel = sel.astype(bf16)
    sel1, sel2 = sel[:10], sel[10:]
    kiota = jnp.arange(K, dtype=jnp.int32).reshape(1, K)

    ci = lambda i: (i, 0)
    cc = lambda i: (0, 0)
    xg, whh_t = pl.pallas_call(
        _precompute_kernel,
        grid=(G1,),
        in_specs=[
            pl.BlockSpec((RC, 2), ci),
            pl.BlockSpec((RC, 1), ci),
            pl.BlockSpec((RC, 1), ci),
            pl.BlockSpec((2, 10), cc),
            pl.BlockSpec((1, 10), cc),
            pl.BlockSpec((1, 10), cc),
            pl.BlockSpec((1, 10), cc),
            pl.BlockSpec((10, K * D), cc),
            pl.BlockSpec((10, K * D), cc),
            pl.BlockSpec((1, K * D), cc),
            pl.BlockSpec((1, K), cc),
            pl.BlockSpec((K * D, H), cc),
            pl.BlockSpec((K, H), cc),
            pl.BlockSpec((4 * H, H), cc),
            pl.BlockSpec((1, 4 * H), cc),
            pl.BlockSpec((WQ, H), ci),
        ],
        out_specs=[
            pl.BlockSpec((RC, 4 * H), ci),
            pl.BlockSpec((H, WQ), lambda i: (0, i)),
        ],
        out_shape=[
            jax.ShapeDtypeStruct((R, 4 * H), bf16),
            jax.ShapeDtypeStruct((H, 4 * H), bf16),
        ],
        scratch_shapes=[
            pltpu.VMEM((H, 4 * H), bf16),
        ],
    )(x1, x2, sk, W_m1.T, b_m1.reshape(1, 10), W_m2.T, b_m2.reshape(1, 10),
      sel1, sel2, expc, kiota, w_flat, benc_b, W_ih, b_gates, W_hh)

    xg3 = xg.reshape(T, B, 4 * H)
    out3, h_t, c_t = pl.pallas_call(
        _recurrent_kernel,
        grid=(T // SPG,),
        in_specs=[
            pl.BlockSpec((SPG, B, 4 * H), lambda t: (t, 0, 0)),
            pl.BlockSpec((B, H), lambda t: (0, 0)),
            pl.BlockSpec((B, H), lambda t: (0, 0)),
            pl.BlockSpec((H, 4 * H), lambda t: (0, 0)),
            pl.BlockSpec((H, K), lambda t: (0, 0)),
            pl.BlockSpec((1, K), lambda t: (0, 0)),
            pl.BlockSpec((1, 4 * H), lambda t: (0, 0)),
        ],
        out_specs=[
            pl.BlockSpec((B, SPG * K), lambda t: (0, t)),
            pl.BlockSpec((B, H), lambda t: (0, 0)),
            pl.BlockSpec((B, H), lambda t: (0, 0)),
        ],
        out_shape=[
            jax.ShapeDtypeStruct((B, T * K), jnp.float32),
            jax.ShapeDtypeStruct((B, H), jnp.float32),
            jax.ShapeDtypeStruct((B, H), jnp.float32),
        ],
        scratch_shapes=[
            pltpu.VMEM((B, H), jnp.float32),
            pltpu.VMEM((B, H), jnp.float32),
        ],
    )(xg3, h0, c0, whh_t, wdec_t, b_dec.reshape(1, K), b_gates)

    output = out3.reshape(B * T, K)
    return (output, h_t, c_t)
